# baseline (speedup 1.0000x reference)
import functools

import jax
import jax.numpy as jnp
from jax import lax
from jax.experimental import pallas as pl
from jax.experimental.pallas import tpu as pltpu
from jax.experimental.pallas import tpu_sc as plsc

B, S, D = 4096, 50, 128
H = D // 2
ROWS = B * S
NC, NS = 2, 16
NW = NC * NS
RPW = ROWS // NW

_mesh = plsc.VectorSubcoreMesh(core_axis_name="c", subcore_axis_name="s")


@functools.partial(
    pl.kernel,
    out_type=jax.ShapeDtypeStruct((ROWS, D), jnp.float32),
    mesh=_mesh,
    scratch_types=[
        pltpu.SemaphoreType.DMA,
    ],
    compiler_params=pltpu.CompilerParams(use_tc_tiling_on_sc=False),
)
def _swap_halves(x_hbm, out_hbm, sem):
    wid = lax.axis_index("s") * NC + lax.axis_index("c")
    base = wid * RPW
    c1 = pltpu.async_copy(
        x_hbm.at[pl.ds(base, RPW), pl.ds(H, H)],
        out_hbm.at[pl.ds(base, RPW), pl.ds(0, H)],
        sem,
    )
    c2 = pltpu.async_copy(
        x_hbm.at[pl.ds(base, RPW), pl.ds(0, H)],
        out_hbm.at[pl.ds(base, RPW), pl.ds(H, H)],
        sem,
    )
    c1.wait()
    c2.wait()


def kernel(x, indices):
    del indices
    out = _swap_halves(x.reshape(ROWS, D))
    return out.reshape(x.shape)


# speedup vs baseline: 7.5171x; 7.5171x over previous
import functools

import jax
import jax.numpy as jnp
from jax import lax
from jax.experimental import pallas as pl
from jax.experimental.pallas import tpu as pltpu
from jax.experimental.pallas import tpu_sc as plsc

B, S, D = 4096, 50, 128
H = D // 2
ROWS = B * S
NC, NS = 2, 16
NW = NC * NS
RPW = ROWS // NW
CH = 400
NCHUNK = RPW // CH

_mesh = plsc.VectorSubcoreMesh(core_axis_name="c", subcore_axis_name="s")


@functools.partial(
    pl.kernel,
    out_type=jax.ShapeDtypeStruct((ROWS, D), jnp.float32),
    mesh=_mesh,
    scratch_types=[
        pltpu.VMEM((CH, D), jnp.float32),
        pltpu.VMEM((CH, D), jnp.float32),
        pltpu.SemaphoreType.DMA,
        pltpu.SemaphoreType.DMA,
        pltpu.SemaphoreType.DMA,
        pltpu.SemaphoreType.DMA,
    ],
    compiler_params=pltpu.CompilerParams(use_tc_tiling_on_sc=False),
)
def _swap_halves(x_hbm, out_hbm, buf0, buf1, in0, in1, out0, out1):
    wid = lax.axis_index("s") * NC + lax.axis_index("c")
    base = wid * RPW
    bufs = (buf0, buf1)
    in_sems = (in0, in1)
    out_sems = (out0, out1)

    def fire_in(i, b):
        pltpu.async_copy(x_hbm.at[pl.ds(base + i * CH, CH), :], bufs[b], in_sems[b])

    def fire_out(i, b):
        r = base + i * CH
        pltpu.async_copy(
            bufs[b].at[:, pl.ds(H, H)], out_hbm.at[pl.ds(r, CH), pl.ds(0, H)],
            out_sems[b],
        )
        pltpu.async_copy(
            bufs[b].at[:, pl.ds(0, H)], out_hbm.at[pl.ds(r, CH), pl.ds(H, H)],
            out_sems[b],
        )

    def wait_in(i, b):
        pltpu.make_async_copy(x_hbm.at[pl.ds(base + i * CH, CH), :], bufs[b],
                              in_sems[b]).wait()

    def wait_out(i, b):
        r = base + i * CH
        pltpu.make_async_copy(
            bufs[b].at[:, pl.ds(H, H)], out_hbm.at[pl.ds(r, CH), pl.ds(0, H)],
            out_sems[b],
        ).wait()
        pltpu.make_async_copy(
            bufs[b].at[:, pl.ds(0, H)], out_hbm.at[pl.ds(r, CH), pl.ds(H, H)],
            out_sems[b],
        ).wait()

    fire_in(0, 0)
    fire_in(1, 1)

    @pl.loop(0, NCHUNK, step=2)
    def _chunks(g):
        for b in range(2):
            i = g + b
            wait_in(i, b)
            fire_out(i, b)
            @pl.when(i + 2 < NCHUNK)
            def _():
                wait_out(i, b)
                fire_in(i + 2, b)

    wait_out(NCHUNK - 2, 0)
    wait_out(NCHUNK - 1, 1)


def kernel(x, indices):
    del indices
    out = _swap_halves(x.reshape(ROWS, D))
    return out.reshape(x.shape)
